# baseline (speedup 1.0000x reference)
import functools

import jax
import jax.numpy as jnp
from jax import lax
from jax.experimental import pallas as pl
from jax.experimental.pallas import tpu as pltpu
from jax.experimental.pallas import tpu_sc as plsc

HEADS = 16
T = 1024
NPOS = T * T
ROWS_PAD = 4096

_info = plsc.get_sparse_core_info()
NC, NS, L = _info.num_cores, _info.num_subcores, _info.num_lanes
NW = NC * NS

POS_PER_W = NPOS // NW
CHUNK = 256
NBUF = 8
CPR = T // CHUNK
N_CHUNKS = POS_PER_W // CHUNK


def _bias_kernel(tbl_hbm, idx_hbm, out_hbm, tbl_v, idx_v, out_v, isem, osem):
    wid = lax.axis_index("s") * NC + lax.axis_index("c")
    c0 = wid * N_CHUNKS

    def idx_copy(slot, c):
        return pltpu.make_async_copy(
            idx_hbm.at[c // CPR, pl.ds((c % CPR) * CHUNK, CHUNK)],
            idx_v.at[slot], isem.at[slot])

    def out_copy(slot, c):
        return pltpu.make_async_copy(
            out_v.at[slot],
            out_hbm.at[:, c // CPR, pl.ds((c % CPR) * CHUNK, CHUNK)],
            osem.at[slot])

    for b in range(NBUF):
        idx_copy(b, c0 + b).start()
    pltpu.sync_copy(tbl_hbm, tbl_v)

    def ring_body(gr, _):
        for b in range(NBUF):
            g = gr * NBUF + b
            c = c0 + g
            idx_copy(b, c).wait()

            @pl.when(gr >= 1)
            def _():
                out_copy(b, c - NBUF).wait()

            @plsc.parallel_loop(0, CHUNK, L, unroll=4)
            def vec_body(p):
                rvec = idx_v[b, pl.ds(p, L)]
                for h in range(HEADS):
                    gathered = plsc.load_gather(tbl_v, [rvec + h * ROWS_PAD])
                    out_v[b, h, pl.ds(p, L)] = gathered

            out_copy(b, c).start()

            @pl.when(g < N_CHUNKS - NBUF)
            def _():
                idx_copy(b, c + NBUF).start()
        return 0

    lax.fori_loop(0, N_CHUNKS // NBUF, ring_body, 0)
    for b in range(NBUF):
        out_copy(b, c0 + N_CHUNKS - NBUF + b).wait()


@jax.jit
def _run(tbl_flat, idx_flat):
    mesh = plsc.VectorSubcoreMesh(core_axis_name="c", subcore_axis_name="s")
    k = functools.partial(
        pl.kernel,
        mesh=mesh,
        out_type=jax.ShapeDtypeStruct((HEADS, T, T), jnp.float32),
        scratch_types=[
            pltpu.VMEM((HEADS * ROWS_PAD,), jnp.float32),
            pltpu.VMEM((NBUF, CHUNK), jnp.int32),
            pltpu.VMEM((NBUF, HEADS, CHUNK), jnp.float32),
            pltpu.SemaphoreType.DMA((NBUF,)),
            pltpu.SemaphoreType.DMA((NBUF,)),
        ],
        compiler_params=pltpu.CompilerParams(
            needs_layout_passes=False,
            disable_bounds_checks=True,
            skip_device_barrier=True,
        ),
    )(_bias_kernel)
    return k(tbl_flat, idx_flat)


def kernel(relative_bias_table, relative_position_index):
    tbl_t = jnp.swapaxes(relative_bias_table, 0, 1)
    tbl_flat = jnp.pad(
        tbl_t, ((0, 0), (0, ROWS_PAD - tbl_t.shape[1]))).reshape(-1)
    return _run(tbl_flat, relative_position_index)


# speedup vs baseline: 1.5687x; 1.5687x over previous
import functools

import jax
import jax.numpy as jnp
from jax import lax
from jax.experimental import pallas as pl
from jax.experimental.pallas import tpu as pltpu
from jax.experimental.pallas import tpu_sc as plsc

HEADS = 16
T = 1024
NPOS = T * T
ROWS_PAD = 4096

_info = plsc.get_sparse_core_info()
NC, NS, L = _info.num_cores, _info.num_subcores, _info.num_lanes
NW = NC * NS

POS_PER_W = NPOS // NW
CHUNK = 512
NBUF = 4
N_CHUNKS = POS_PER_W // CHUNK


def _bias_kernel(tbl_hbm, idx_hbm, out_hbm, tbl_v, idx_v, out_v, isem, osem):
    wid = lax.axis_index("s") * NC + lax.axis_index("c")
    c0 = wid * N_CHUNKS

    def idx_copy(slot, c):
        return pltpu.make_async_copy(
            idx_hbm.at[c // 2, pl.ds((c % 2) * CHUNK, CHUNK)],
            idx_v.at[slot], isem.at[slot])

    def out_copy(slot, c):
        return pltpu.make_async_copy(
            out_v.at[slot],
            out_hbm.at[:, c // 2, pl.ds((c % 2) * CHUNK, CHUNK)],
            osem.at[slot])

    for b in range(NBUF):
        idx_copy(b, c0 + b).start()
    pltpu.sync_copy(tbl_hbm, tbl_v)

    def ring_body(gr, _):
        for b in range(NBUF):
            g = gr * NBUF + b
            c = c0 + g
            idx_copy(b, c).wait()

            @pl.when(gr >= 1)
            def _():
                out_copy(b, c - NBUF).wait()

            @plsc.parallel_loop(0, CHUNK, L, unroll=4)
            def vec_body(p):
                rvec = idx_v[b, pl.ds(p, L)]
                for h in range(HEADS):
                    gathered = plsc.load_gather(tbl_v, [rvec + h * ROWS_PAD])
                    out_v[b, h, pl.ds(p, L)] = gathered

            out_copy(b, c).start()

            @pl.when(g < N_CHUNKS - NBUF)
            def _():
                idx_copy(b, c + NBUF).start()
        return 0

    lax.fori_loop(0, N_CHUNKS // NBUF, ring_body, 0)
    for b in range(NBUF):
        out_copy(b, c0 + N_CHUNKS - NBUF + b).wait()


@jax.jit
def _run(tbl_flat, idx_flat):
    mesh = plsc.VectorSubcoreMesh(core_axis_name="c", subcore_axis_name="s")
    k = functools.partial(
        pl.kernel,
        mesh=mesh,
        out_type=jax.ShapeDtypeStruct((HEADS, T, T), jnp.float32),
        scratch_types=[
            pltpu.VMEM((HEADS * ROWS_PAD,), jnp.float32),
            pltpu.VMEM((NBUF, CHUNK), jnp.int32),
            pltpu.VMEM((NBUF, HEADS, CHUNK), jnp.float32),
            pltpu.SemaphoreType.DMA((NBUF,)),
            pltpu.SemaphoreType.DMA((NBUF,)),
        ],
        compiler_params=pltpu.CompilerParams(
            needs_layout_passes=False,
            disable_bounds_checks=True,
            skip_device_barrier=True,
        ),
    )(_bias_kernel)
    return k(tbl_flat, idx_flat)


def kernel(relative_bias_table, relative_position_index):
    tbl_t = jnp.swapaxes(relative_bias_table, 0, 1)
    tbl_flat = jnp.pad(
        tbl_t, ((0, 0), (0, ROWS_PAD - tbl_t.shape[1]))).reshape(-1)
    return _run(tbl_flat, relative_position_index)
